# Initial kernel scaffold; baseline (speedup 1.0000x reference)
import jax
import jax.numpy as jnp
from jax.experimental import pallas as pl


def kernel(features, labels, prototypes):
    raise NotImplementedError("write your pallas kernel here")


# speedup vs baseline: 968.7881x; 968.7881x over previous
import functools

import jax
import jax.numpy as jnp
from jax import lax
from jax.experimental import pallas as pl
from jax.experimental.pallas import tpu as pltpu
from jax.experimental.pallas import tpu_sc as plsc

N_CLS = 1000
D = 128
B = 16384
PROTO_M = 0.999
TEMP = 0.1
BASE_TEMP = 0.1

NCLS_PAD = 1024
NW = 32
K = NCLS_PAD // NW
NV = D // 16
CHUNK = 32
IDXC = CHUNK + 16
PERM_LEN = B + 64
OFF_SLAB = K + 17
OFF_SLAB_PAD = 48
OFF_LEN = NCLS_PAD - K + OFF_SLAB_PAD


def _ema_kernel(feat_hbm, perm_hbm, off_hbm, protos_hbm, out_hbm,
                protos_v, off_v, idx_v, rows_v, sem):
    wid = lax.axis_index("s") * 2 + lax.axis_index("c")
    c0 = wid * K
    pltpu.sync_copy(protos_hbm.at[pl.ds(c0, K)], protos_v)
    pltpu.sync_copy(off_hbm.at[pl.ds(pl.multiple_of(c0, 16), OFF_SLAB_PAD)],
                    off_v)
    lane = lax.iota(jnp.int32, 16)
    bfly = [lax.bitwise_xor(lane, jnp.int32(s)) for s in (8, 4, 2, 1)]

    def class_body(i, carry):
        o = off_v[pl.ds(i, 16)][0]
        k = off_v[pl.ds(i + 1, 16)][0] - o
        p = [protos_v[i, pl.ds(v * 16, 16)] for v in range(NV)]

        def chunk_body(t, p):
            base = o + t * CHUNK
            al = pl.multiple_of(lax.bitwise_and(base, -16), 16)
            sh = base - al
            pltpu.sync_copy(perm_hbm.at[pl.ds(al, IDXC)], idx_v)
            pltpu.async_copy(feat_hbm.at[idx_v], rows_v, sem).wait()
            kc = jnp.minimum(CHUNK, k - t * CHUNK)

            def samp_body(j, p):
                r = sh + j
                q = [PROTO_M * p[v]
                     + (1.0 - PROTO_M) * rows_v[r, pl.ds(v * 16, 16)]
                     for v in range(NV)]
                acc = q[0] * q[0]
                for v in range(1, NV):
                    acc = acc + q[v] * q[v]
                for bf in bfly:
                    acc = acc + acc.at[bf].get(mode="promise_in_bounds")
                n2v = jnp.maximum(acc, jnp.float32(1e-24))
                y = plsc.bitcast(
                    jnp.int32(0x5F3759DF)
                    - lax.shift_right_arithmetic(
                        plsc.bitcast(n2v, jnp.int32), jnp.int32(1)),
                    jnp.float32)
                h = jnp.float32(0.5) * n2v
                for _ in range(3):
                    y = y * (jnp.float32(1.5) - h * y * y)
                return [q[v] * y for v in range(NV)]

            return lax.fori_loop(0, kc, samp_body, p)

        nchunks = lax.div(k + (CHUNK - 1), CHUNK)
        p = lax.fori_loop(0, nchunks, chunk_body, p)
        for v in range(NV):
            protos_v[i, pl.ds(v * 16, 16)] = p[v]
        return carry

    lax.fori_loop(0, K, class_body, 0)
    pltpu.sync_copy(protos_v, out_hbm.at[pl.ds(c0, K)])


def _loss_kernel(p_ref, o_ref):
    p = p_ref[...]
    s = lax.dot_general(p, p, (((1,), (1,)), ((), ())),
                        preferred_element_type=jnp.float32) * (1.0 / TEMP)
    row = lax.broadcasted_iota(jnp.int32, (NCLS_PAD, NCLS_PAD), 0)
    col = lax.broadcasted_iota(jnp.int32, (NCLS_PAD, NCLS_PAD), 1)
    neg = jnp.logical_and(col < N_CLS, col != row)
    e = jnp.where(neg, jnp.exp(s), 0.0)
    rs = jnp.sum(e, axis=1)
    mpn = jnp.log(rs / jnp.float32(N_CLS - 1))
    rvalid = lax.broadcasted_iota(jnp.int32, (NCLS_PAD, 1), 0) < N_CLS
    total = jnp.sum(jnp.where(rvalid[:, 0], mpn, 0.0))
    loss = (TEMP / BASE_TEMP) * total / jnp.float32(N_CLS)
    o_ref[...] = jnp.full((8, 128), loss, jnp.float32)


@jax.jit
def kernel(features, labels, prototypes):
    labels = labels.astype(jnp.int32)
    perm = jnp.argsort(labels, stable=True).astype(jnp.int32)
    perm = jnp.pad(perm, (0, PERM_LEN - B))
    counts = jnp.bincount(labels, length=NCLS_PAD).astype(jnp.int32)
    off = jnp.concatenate(
        [jnp.zeros((1,), jnp.int32), jnp.cumsum(counts, dtype=jnp.int32)])
    off = jnp.pad(off, (0, OFF_LEN - off.shape[0]))
    protos_pad = jnp.pad(prototypes, ((0, NCLS_PAD - N_CLS), (0, 0)))

    ema = pl.kernel(
        _ema_kernel,
        out_type=jax.ShapeDtypeStruct((NCLS_PAD, D), jnp.float32),
        mesh=plsc.VectorSubcoreMesh(core_axis_name="c", subcore_axis_name="s"),
        compiler_params=pltpu.CompilerParams(needs_layout_passes=False),
        scratch_types=[
            pltpu.VMEM((K, D), jnp.float32),
            pltpu.VMEM((OFF_SLAB_PAD,), jnp.int32),
            pltpu.VMEM((IDXC,), jnp.int32),
            pltpu.VMEM((IDXC, D), jnp.float32),
            pltpu.SemaphoreType.DMA,
        ],
    )
    protos_new = ema(features, perm, off, protos_pad)

    loss2d = pl.pallas_call(
        _loss_kernel,
        out_shape=jax.ShapeDtypeStruct((8, 128), jnp.float32),
    )(protos_new)
    return loss2d[0, 0]


# speedup vs baseline: 1243.0494x; 1.2831x over previous
import functools

import jax
import jax.numpy as jnp
from jax import lax
from jax.experimental import pallas as pl
from jax.experimental.pallas import tpu as pltpu
from jax.experimental.pallas import tpu_sc as plsc

N_CLS = 1000
D = 128
B = 16384
PROTO_M = 0.999
TEMP = 0.1
BASE_TEMP = 0.1

NCLS_PAD = 1024
NW = 32
K = NCLS_PAD // NW
NV = D // 16
CH = 256
PERM_LEN = B + 2 * CH
OFF_SLAB = 48
OFF_LEN = NCLS_PAD - K + OFF_SLAB


def _ema_kernel(feat_hbm, perm_hbm, lbl_hbm, off_hbm, protos_hbm, out_hbm,
                protos_v, off_v, idx0, idx1, lbl0, lbl1, rows0, rows1,
                sem0, sem1):
    wid = lax.axis_index("s") * 2 + lax.axis_index("c")
    c0 = wid * K
    pltpu.sync_copy(protos_hbm.at[pl.ds(c0, K)], protos_v)
    pltpu.sync_copy(off_hbm.at[pl.ds(pl.multiple_of(c0, 16), OFF_SLAB)],
                    off_v)
    lane = lax.iota(jnp.int32, 16)
    bfly = [lax.bitwise_xor(lane, jnp.int32(s)) for s in (8, 4, 2, 1)]
    sems = (sem0, sem1)
    idxs = (idx0, idx1)
    lbls = (lbl0, lbl1)
    rows = (rows0, rows1)

    s0 = off_v[pl.ds(0, 16)][0]
    s1 = off_v[pl.ds(K, 16)][0]
    al = pl.multiple_of(lax.bitwise_and(s0, -16), 16)
    nwin = lax.div(s1 - al + (CH - 1), CH)

    def win_start(t):
        return pl.multiple_of(al + t * CH, 16)

    def prefetch(t, buf):
        pltpu.sync_copy(perm_hbm.at[pl.ds(win_start(t), CH)], idxs[buf])
        pltpu.sync_copy(lbl_hbm.at[pl.ds(win_start(t), CH + 16)], lbls[buf])
        pltpu.async_copy(feat_hbm.at[idxs[buf]], rows[buf], sems[buf])

    @pl.when(nwin > 0)
    def _():
        prefetch(0, 0)

    def do_window(t, buf, cur_p):
        @pl.when(t + 1 < nwin)
        def _():
            prefetch(t + 1, 1 - buf)

        @pl.when(t < nwin)
        def _():
            pltpu.make_async_copy(feat_hbm.at[idxs[buf]],
                                  rows[buf], sems[buf]).wait()

        lo = jnp.maximum(s0 - al - t * CH, 0)
        hi = jnp.minimum(s1 - al - t * CH, CH)

        def samp_body(r, cur_p):
            cur = cur_p[0]
            l = lbls[buf][pl.ds(r, 16)][0]

            def sw_true(cur, *p):
                for v in range(NV):
                    protos_v[cur - c0, pl.ds(v * 16, 16)] = p[v]
                return (l,) + tuple(
                    protos_v[l - c0, pl.ds(v * 16, 16)] for v in range(NV))

            def sw_false(cur, *p):
                return (cur,) + tuple(p)

            cur_p = lax.cond(l != cur, sw_true, sw_false, *cur_p)
            p = cur_p[1:]
            q = [PROTO_M * p[v]
                 + (1.0 - PROTO_M) * rows[buf][r, pl.ds(v * 16, 16)]
                 for v in range(NV)]
            d = [q[2 * v] * q[2 * v] + q[2 * v + 1] * q[2 * v + 1]
                 for v in range(NV // 2)]
            d = [d[0] + d[1], d[2] + d[3]]
            acc = d[0] + d[1]
            for bf in bfly:
                acc = acc + acc.at[bf].get(mode="promise_in_bounds")
            n2v = jnp.maximum(acc, jnp.float32(1e-24))
            y = plsc.bitcast(
                jnp.int32(0x5F3759DF)
                - lax.shift_right_arithmetic(
                    plsc.bitcast(n2v, jnp.int32), jnp.int32(1)),
                jnp.float32)
            h = jnp.float32(0.5) * n2v
            for _ in range(3):
                y = y * (jnp.float32(1.5) - h * y * y)
            return (cur_p[0],) + tuple(q[v] * y for v in range(NV))

        return lax.fori_loop(lo, hi, samp_body, cur_p)

    def pair_body(g, cur_p):
        cur_p = do_window(2 * g, 0, cur_p)
        cur_p = do_window(2 * g + 1, 1, cur_p)
        return cur_p

    cur_p = (c0,) + tuple(protos_v[0, pl.ds(v * 16, 16)] for v in range(NV))
    cur_p = lax.fori_loop(0, lax.div(nwin + 1, 2), pair_body, cur_p)
    for v in range(NV):
        protos_v[cur_p[0] - c0, pl.ds(v * 16, 16)] = cur_p[1 + v]
    pltpu.sync_copy(protos_v, out_hbm.at[pl.ds(c0, K)])


def _loss_kernel(p_ref, o_ref):
    p = p_ref[...]
    s = lax.dot_general(p, p, (((1,), (1,)), ((), ())),
                        preferred_element_type=jnp.float32) * (1.0 / TEMP)
    row = lax.broadcasted_iota(jnp.int32, (NCLS_PAD, NCLS_PAD), 0)
    col = lax.broadcasted_iota(jnp.int32, (NCLS_PAD, NCLS_PAD), 1)
    neg = jnp.logical_and(col < N_CLS, col != row)
    e = jnp.where(neg, jnp.exp(s), 0.0)
    rs = jnp.sum(e, axis=1)
    mpn = jnp.log(rs / jnp.float32(N_CLS - 1))
    rvalid = lax.broadcasted_iota(jnp.int32, (NCLS_PAD, 1), 0) < N_CLS
    total = jnp.sum(jnp.where(rvalid[:, 0], mpn, 0.0))
    loss = (TEMP / BASE_TEMP) * total / jnp.float32(N_CLS)
    o_ref[...] = jnp.full((8, 128), loss, jnp.float32)


@jax.jit
def kernel(features, labels, prototypes):
    labels = labels.astype(jnp.int32)
    perm = jnp.argsort(labels, stable=True).astype(jnp.int32)
    lbl_sorted = labels[perm]
    perm = jnp.pad(perm, (0, PERM_LEN - B))
    lbl_sorted = jnp.pad(lbl_sorted, (0, PERM_LEN - B))
    counts = jnp.bincount(labels, length=NCLS_PAD).astype(jnp.int32)
    off = jnp.concatenate(
        [jnp.zeros((1,), jnp.int32), jnp.cumsum(counts, dtype=jnp.int32)])
    off = jnp.pad(off, (0, OFF_LEN - off.shape[0]))
    protos_pad = jnp.pad(prototypes, ((0, NCLS_PAD - N_CLS), (0, 0)))

    ema = pl.kernel(
        _ema_kernel,
        out_type=jax.ShapeDtypeStruct((NCLS_PAD, D), jnp.float32),
        mesh=plsc.VectorSubcoreMesh(core_axis_name="c", subcore_axis_name="s"),
        compiler_params=pltpu.CompilerParams(needs_layout_passes=False),
        scratch_types=[
            pltpu.VMEM((K, D), jnp.float32),
            pltpu.VMEM((OFF_SLAB,), jnp.int32),
            pltpu.VMEM((CH,), jnp.int32),
            pltpu.VMEM((CH,), jnp.int32),
            pltpu.VMEM((CH + 16,), jnp.int32),
            pltpu.VMEM((CH + 16,), jnp.int32),
            pltpu.VMEM((CH, D), jnp.float32),
            pltpu.VMEM((CH, D), jnp.float32),
            pltpu.SemaphoreType.DMA,
            pltpu.SemaphoreType.DMA,
        ],
    )
    protos_new = ema(features, perm, lbl_sorted, off, protos_pad)

    loss2d = pl.pallas_call(
        _loss_kernel,
        out_shape=jax.ShapeDtypeStruct((8, 128), jnp.float32),
    )(protos_new)
    return loss2d[0, 0]


# speedup vs baseline: 1356.9939x; 1.0917x over previous
import functools

import jax
import jax.numpy as jnp
from jax import lax
from jax.experimental import pallas as pl
from jax.experimental.pallas import tpu as pltpu
from jax.experimental.pallas import tpu_sc as plsc

N_CLS = 1000
D = 128
B = 16384
PROTO_M = 0.999
TEMP = 0.1
BASE_TEMP = 0.1

NCLS_PAD = 1024
NW = 32
K = NCLS_PAD // NW
NV = D // 16
CH = 256
PERM_LEN = B + 2 * CH
OFF_SLAB = 64
OFF_LEN = NCLS_PAD - K + OFF_SLAB


def _ema_kernel(feat_hbm, perm_hbm, off_hbm, protos_hbm, out_hbm,
                protos_v, off_v, idx0, idx1, rows0, rows1, sem0, sem1):
    wid = lax.axis_index("s") * 2 + lax.axis_index("c")
    c0 = wid * K
    pltpu.sync_copy(protos_hbm.at[pl.ds(c0, K)], protos_v)
    pltpu.sync_copy(off_hbm.at[pl.ds(pl.multiple_of(c0, 16), OFF_SLAB)],
                    off_v)
    lane = lax.iota(jnp.int32, 16)
    bfly = [lax.bitwise_xor(lane, jnp.int32(s)) for s in (8, 4, 2, 1)]
    sems = (sem0, sem1)
    idxs = (idx0, idx1)
    rows = (rows0, rows1)

    def off_at(i):
        return off_v[pl.ds(i, 16)][0]

    s0 = off_at(0)
    s1 = off_at(K)
    al = pl.multiple_of(lax.bitwise_and(s0, -16), 16)
    nwin = lax.div(s1 - al + (CH - 1), CH)

    def win_start(t):
        return pl.multiple_of(al + t * CH, 16)

    def prefetch(t, buf):
        pltpu.sync_copy(perm_hbm.at[pl.ds(win_start(t), CH)], idxs[buf])
        pltpu.async_copy(feat_hbm.at[idxs[buf]], rows[buf], sems[buf])

    @pl.when(nwin > 0)
    def _():
        prefetch(0, 0)

    def do_window(t, buf, cur_p):
        @pl.when(t + 1 < nwin)
        def _():
            prefetch(t + 1, 1 - buf)

        @pl.when(t < nwin)
        def _():
            pltpu.make_async_copy(feat_hbm.at[idxs[buf]],
                                  rows[buf], sems[buf]).wait()

        base = al + t * CH
        lo = jnp.maximum(s0 - base, 0)
        hi = jnp.minimum(s1 - base, CH)

        def samp_body(r, cur_p):
            j = base + r

            def sw_true(ci, bnd, *p):
                for v in range(NV):
                    protos_v[ci, pl.ds(v * 16, 16)] = p[v]
                ci = lax.while_loop(lambda a: off_at(a + 1) <= j,
                                    lambda a: a + 1, ci)
                return (ci, off_at(ci + 1)) + tuple(
                    protos_v[ci, pl.ds(v * 16, 16)] for v in range(NV))

            def sw_false(ci, bnd, *p):
                return (ci, bnd) + tuple(p)

            cur_p = lax.cond(j >= cur_p[1], sw_true, sw_false, *cur_p)
            p = cur_p[2:]
            q = [PROTO_M * p[v]
                 + (1.0 - PROTO_M) * rows[buf][r, pl.ds(v * 16, 16)]
                 for v in range(NV)]
            d = [q[2 * v] * q[2 * v] + q[2 * v + 1] * q[2 * v + 1]
                 for v in range(NV // 2)]
            d = [d[0] + d[1], d[2] + d[3]]
            acc = d[0] + d[1]
            for bf in bfly:
                acc = acc + acc.at[bf].get(mode="promise_in_bounds")
            n2v = jnp.maximum(acc, jnp.float32(1e-24))
            y = plsc.bitcast(
                jnp.int32(0x5F3759DF)
                - lax.shift_right_arithmetic(
                    plsc.bitcast(n2v, jnp.int32), jnp.int32(1)),
                jnp.float32)
            h = jnp.float32(0.5) * n2v
            for _ in range(3):
                y = y * (jnp.float32(1.5) - h * y * y)
            return cur_p[:2] + tuple(q[v] * y for v in range(NV))

        return lax.fori_loop(lo, hi, samp_body, cur_p)

    def pair_body(g, cur_p):
        cur_p = do_window(2 * g, 0, cur_p)
        cur_p = do_window(2 * g + 1, 1, cur_p)
        return cur_p

    ci0 = lax.while_loop(
        lambda a: jnp.logical_and(a < K, off_at(a + 1) <= s0),
        lambda a: a + 1, 0)
    ci0 = jnp.minimum(ci0, K - 1)
    cur_p = (ci0, off_at(ci0 + 1)) + tuple(
        protos_v[ci0, pl.ds(v * 16, 16)] for v in range(NV))
    cur_p = lax.fori_loop(0, lax.div(nwin + 1, 2), pair_body, cur_p)

    @pl.when(s1 > s0)
    def _():
        for v in range(NV):
            protos_v[cur_p[0], pl.ds(v * 16, 16)] = cur_p[2 + v]

    pltpu.sync_copy(protos_v, out_hbm.at[pl.ds(c0, K)])


def _loss_kernel(p_ref, o_ref):
    p = p_ref[...]
    s = lax.dot_general(p, p, (((1,), (1,)), ((), ())),
                        preferred_element_type=jnp.float32) * (1.0 / TEMP)
    row = lax.broadcasted_iota(jnp.int32, (NCLS_PAD, NCLS_PAD), 0)
    col = lax.broadcasted_iota(jnp.int32, (NCLS_PAD, NCLS_PAD), 1)
    neg = jnp.logical_and(col < N_CLS, col != row)
    e = jnp.where(neg, jnp.exp(s), 0.0)
    rs = jnp.sum(e, axis=1)
    mpn = jnp.log(rs / jnp.float32(N_CLS - 1))
    rvalid = lax.broadcasted_iota(jnp.int32, (NCLS_PAD, 1), 0) < N_CLS
    total = jnp.sum(jnp.where(rvalid[:, 0], mpn, 0.0))
    loss = (TEMP / BASE_TEMP) * total / jnp.float32(N_CLS)
    o_ref[...] = jnp.full((8, 128), loss, jnp.float32)


@jax.jit
def kernel(features, labels, prototypes):
    labels = labels.astype(jnp.int32)
    perm = jnp.argsort(labels, stable=True).astype(jnp.int32)
    perm = jnp.pad(perm, (0, PERM_LEN - B))
    counts = jnp.bincount(labels, length=NCLS_PAD).astype(jnp.int32)
    off = jnp.concatenate(
        [jnp.zeros((1,), jnp.int32), jnp.cumsum(counts, dtype=jnp.int32)])
    off = jnp.pad(off, (0, OFF_LEN - off.shape[0]),
                  constant_values=jnp.int32(B))
    protos_pad = jnp.pad(prototypes, ((0, NCLS_PAD - N_CLS), (0, 0)))

    ema = pl.kernel(
        _ema_kernel,
        out_type=jax.ShapeDtypeStruct((NCLS_PAD, D), jnp.float32),
        mesh=plsc.VectorSubcoreMesh(core_axis_name="c", subcore_axis_name="s"),
        compiler_params=pltpu.CompilerParams(needs_layout_passes=False),
        scratch_types=[
            pltpu.VMEM((K, D), jnp.float32),
            pltpu.VMEM((OFF_SLAB,), jnp.int32),
            pltpu.VMEM((CH,), jnp.int32),
            pltpu.VMEM((CH,), jnp.int32),
            pltpu.VMEM((CH, D), jnp.float32),
            pltpu.VMEM((CH, D), jnp.float32),
            pltpu.SemaphoreType.DMA,
            pltpu.SemaphoreType.DMA,
        ],
    )
    protos_new = ema(features, perm, off, protos_pad)

    loss2d = pl.pallas_call(
        _loss_kernel,
        out_shape=jax.ShapeDtypeStruct((8, 128), jnp.float32),
    )(protos_new)
    return loss2d[0, 0]


# speedup vs baseline: 1591.3082x; 1.1727x over previous
import functools

import jax
import jax.numpy as jnp
from jax import lax
from jax.experimental import pallas as pl
from jax.experimental.pallas import tpu as pltpu
from jax.experimental.pallas import tpu_sc as plsc

N_CLS = 1000
D = 128
B = 16384
PROTO_M = 0.999
TEMP = 0.1
BASE_TEMP = 0.1

NCLS_PAD = 1024
NW = 32
K = NCLS_PAD // NW
NV = D // 16
CH = 256
NT = 16
SLICE = B // NT
NVREG = SLICE // 16
QLEN = B + 2 * CH
BASE_LEN = NCLS_PAD + 64


def _ema_kernel(feat_hbm, lbl_hbm, protos_hbm, out_hbm,
                protos_v, lblv, cntv, matv, basev, curv, posb, valb,
                idx0, idx1, rows0, rows1, cntmat_sh, queue_sh,
                sem0, sem1):
    sid = lax.axis_index("s")
    wid = sid * 2 + lax.axis_index("c")
    c0 = wid * K
    lane = lax.iota(jnp.int32, 16)
    zeros16 = jnp.zeros((16,), jnp.int32)
    ones16 = jnp.ones((16,), jnp.int32)

    pltpu.sync_copy(lbl_hbm.at[pl.ds(sid * SLICE, SLICE)], lblv)

    def zero_body(i, _):
        cntv[pl.ds(i * 16, 16)] = zeros16
        return _
    lax.fori_loop(0, NCLS_PAD // 16, zero_body, 0)

    def hist_body(i, _):
        lv = lblv[pl.ds(i * 16, 16)]
        plsc.addupdate_scatter(cntv, [lv], ones16)
        return _
    lax.fori_loop(0, NVREG, hist_body, 0)

    pltpu.sync_copy(cntv, cntmat_sh.at[sid])
    plsc.subcore_barrier()
    pltpu.sync_copy(cntmat_sh, matv)

    def colsum_body(j, _):
        tot = zeros16
        pre = zeros16
        for t in range(NT):
            row = matv[t, pl.ds(j * 16, 16)]
            tot = tot + row
            pre = pre + row * (jnp.int32(t) < sid).astype(jnp.int32)
        cntv[pl.ds(j * 16, 16)] = tot
        curv[pl.ds(j * 16, 16)] = pre
        return _
    lax.fori_loop(0, NCLS_PAD // 16, colsum_body, 0)

    def cumsum_body(j, run):
        chv = cntv[pl.ds(j * 16, 16)]
        inc = plsc.cumsum(chv)
        ex = inc - chv + jnp.full((16,), run, jnp.int32)
        basev[pl.ds(j * 16, 16)] = ex
        curv[pl.ds(j * 16, 16)] = curv[pl.ds(j * 16, 16)] + ex
        return run + inc[15]
    total = lax.fori_loop(0, NCLS_PAD // 16, cumsum_body, jnp.int32(0))
    for j in range(NCLS_PAD // 16, BASE_LEN // 16):
        basev[pl.ds(j * 16, 16)] = jnp.full((16,), total, jnp.int32)

    def place_body(i, _):
        lv = lblv[pl.ds(i * 16, 16)]
        cnt, _last = plsc.scan_count(lv)
        g = plsc.load_gather(curv, [lv])
        pos = g + cnt - 1
        r = lax.div(i, 8)
        col = 16 * lax.rem(i, 8)
        posb[r, pl.ds(col, 16)] = pos
        valb[r, pl.ds(col, 16)] = sid * SLICE + i * 16 + lane
        plsc.addupdate_scatter(curv, [lv], ones16)
        return _
    lax.fori_loop(0, NVREG, place_body, 0)

    for jj in range(8):
        pltpu.async_copy(valb.at[jj], queue_sh.at[posb.at[jj]], sem0)
    for jj in range(8):
        pltpu.make_async_copy(valb.at[jj], queue_sh.at[posb.at[jj]],
                              sem0).wait()

    @pl.when(sid == 0)
    def _():
        def zq_body(i, _):
            idx0[pl.ds(i * 16, 16)] = zeros16
            return _
        lax.fori_loop(0, CH // 16, zq_body, 0)
        pltpu.sync_copy(idx0, queue_sh.at[pl.ds(B, CH)])
        pltpu.sync_copy(idx0, queue_sh.at[pl.ds(B + CH, CH)])
    plsc.subcore_barrier()

    pltpu.sync_copy(protos_hbm.at[pl.ds(c0, K)], protos_v)
    bfly = [lax.bitwise_xor(lane, jnp.int32(s)) for s in (8, 4, 2, 1)]
    sems = (sem0, sem1)
    idxs = (idx0, idx1)
    rows = (rows0, rows1)

    def off_at(i):
        return basev[pl.ds(c0 + i, 16)][0]

    s0 = off_at(0)
    s1 = off_at(K)
    al = pl.multiple_of(lax.bitwise_and(s0, -16), 16)
    nwin = lax.div(s1 - al + (CH - 1), CH)

    def win_start(t):
        return pl.multiple_of(al + t * CH, 16)

    def prefetch(t, buf):
        pltpu.sync_copy(queue_sh.at[pl.ds(win_start(t), CH)], idxs[buf])
        pltpu.async_copy(feat_hbm.at[idxs[buf]], rows[buf], sems[buf])

    @pl.when(nwin > 0)
    def _():
        prefetch(0, 0)

    def do_window(t, buf, cur_p):
        @pl.when(t + 1 < nwin)
        def _():
            prefetch(t + 1, 1 - buf)

        @pl.when(t < nwin)
        def _():
            pltpu.make_async_copy(feat_hbm.at[idxs[buf]],
                                  rows[buf], sems[buf]).wait()

        base = al + t * CH
        lo = jnp.maximum(s0 - base, 0)
        hi = jnp.minimum(s1 - base, CH)

        def samp_body(r, cur_p):
            j = base + r

            def sw_true(ci, bnd, *p):
                for v in range(NV):
                    protos_v[ci, pl.ds(v * 16, 16)] = p[v]
                ci = lax.while_loop(lambda a: off_at(a + 1) <= j,
                                    lambda a: a + 1, ci)
                return (ci, off_at(ci + 1)) + tuple(
                    protos_v[ci, pl.ds(v * 16, 16)] for v in range(NV))

            def sw_false(ci, bnd, *p):
                return (ci, bnd) + tuple(p)

            cur_p = lax.cond(j >= cur_p[1], sw_true, sw_false, *cur_p)
            p = cur_p[2:]
            q = [PROTO_M * p[v]
                 + (1.0 - PROTO_M) * rows[buf][r, pl.ds(v * 16, 16)]
                 for v in range(NV)]
            d = [q[2 * v] * q[2 * v] + q[2 * v + 1] * q[2 * v + 1]
                 for v in range(NV // 2)]
            d = [d[0] + d[1], d[2] + d[3]]
            acc = d[0] + d[1]
            for bf in bfly:
                acc = acc + acc.at[bf].get(mode="promise_in_bounds")
            n2v = jnp.maximum(acc, jnp.float32(1e-24))
            y = plsc.bitcast(
                jnp.int32(0x5F3759DF)
                - lax.shift_right_arithmetic(
                    plsc.bitcast(n2v, jnp.int32), jnp.int32(1)),
                jnp.float32)
            h = jnp.float32(0.5) * n2v
            for _ in range(3):
                y = y * (jnp.float32(1.5) - h * y * y)
            return cur_p[:2] + tuple(q[v] * y for v in range(NV))

        return lax.fori_loop(lo, hi, samp_body, cur_p)

    def pair_body(g, cur_p):
        cur_p = do_window(2 * g, 0, cur_p)
        cur_p = do_window(2 * g + 1, 1, cur_p)
        return cur_p

    ci0 = lax.while_loop(
        lambda a: jnp.logical_and(a < K, off_at(a + 1) <= s0),
        lambda a: a + 1, 0)
    ci0 = jnp.minimum(ci0, K - 1)
    cur_p = (ci0, off_at(ci0 + 1)) + tuple(
        protos_v[ci0, pl.ds(v * 16, 16)] for v in range(NV))
    cur_p = lax.fori_loop(0, lax.div(nwin + 1, 2), pair_body, cur_p)

    @pl.when(s1 > s0)
    def _():
        for v in range(NV):
            protos_v[cur_p[0], pl.ds(v * 16, 16)] = cur_p[2 + v]

    pltpu.sync_copy(protos_v, out_hbm.at[pl.ds(c0, K)])


def _loss_kernel(p_ref, o_ref):
    p = p_ref[...]
    s = lax.dot_general(p, p, (((1,), (1,)), ((), ())),
                        preferred_element_type=jnp.float32) * (1.0 / TEMP)
    row = lax.broadcasted_iota(jnp.int32, (NCLS_PAD, NCLS_PAD), 0)
    col = lax.broadcasted_iota(jnp.int32, (NCLS_PAD, NCLS_PAD), 1)
    neg = jnp.logical_and(col < N_CLS, col != row)
    e = jnp.where(neg, jnp.exp(s), 0.0)
    rs = jnp.sum(e, axis=1)
    mpn = jnp.log(rs / jnp.float32(N_CLS - 1))
    rvalid = lax.broadcasted_iota(jnp.int32, (NCLS_PAD, 1), 0) < N_CLS
    total = jnp.sum(jnp.where(rvalid[:, 0], mpn, 0.0))
    loss = (TEMP / BASE_TEMP) * total / jnp.float32(N_CLS)
    o_ref[...] = jnp.full((8, 128), loss, jnp.float32)


@jax.jit
def kernel(features, labels, prototypes):
    labels = labels.astype(jnp.int32)
    protos_pad = jnp.pad(prototypes, ((0, NCLS_PAD - N_CLS), (0, 0)))

    ema = pl.kernel(
        _ema_kernel,
        out_type=jax.ShapeDtypeStruct((NCLS_PAD, D), jnp.float32),
        mesh=plsc.VectorSubcoreMesh(core_axis_name="c", subcore_axis_name="s"),
        compiler_params=pltpu.CompilerParams(needs_layout_passes=False),
        scratch_types=[
            pltpu.VMEM((K, D), jnp.float32),
            pltpu.VMEM((SLICE,), jnp.int32),
            pltpu.VMEM((NCLS_PAD,), jnp.int32),
            pltpu.VMEM((NT, NCLS_PAD), jnp.int32),
            pltpu.VMEM((BASE_LEN,), jnp.int32),
            pltpu.VMEM((NCLS_PAD,), jnp.int32),
            pltpu.VMEM((8, 128), jnp.int32),
            pltpu.VMEM((8, 128), jnp.int32),
            pltpu.VMEM((CH,), jnp.int32),
            pltpu.VMEM((CH,), jnp.int32),
            pltpu.VMEM((CH, D), jnp.float32),
            pltpu.VMEM((CH, D), jnp.float32),
            pltpu.VMEM_SHARED((NT, NCLS_PAD), jnp.int32),
            pltpu.VMEM_SHARED((QLEN,), jnp.int32),
            pltpu.SemaphoreType.DMA,
            pltpu.SemaphoreType.DMA,
        ],
    )
    protos_new = ema(features, labels, protos_pad)

    loss2d = pl.pallas_call(
        _loss_kernel,
        out_shape=jax.ShapeDtypeStruct((8, 128), jnp.float32),
    )(protos_new)
    return loss2d[0, 0]
